# baseline (speedup 1.0000x reference)
import jax
import jax.numpy as jnp
from jax import lax
from jax.experimental import pallas as pl
from jax.experimental.pallas import tpu as pltpu, tpu_sc as plsc

B = 16384
D = 64
NNEG = 20
NC = 2
NS = 16
NW = NC * NS
BPW = B // NW
CH = 16
NCHUNK = BPW // CH
L = 16
GPC = CH // L
DP = 2 * D
JH = 5
NWAVE = NNEG // JH


def _body(in_table, out_table, in_idx, ctx_idx, neg_idx, pos_out, neg_out,
          in_idx_v, ctx_idx_v, neg_raw_v, neg_idx_t, npar_t, in_rows,
          pos_rows, neg_rows, pos_v, neg_v, sem):
    wid = lax.axis_index("s") * NC + lax.axis_index("c")

    pltpu.sync_copy(in_idx.at[pl.ds(wid * BPW, BPW)], in_idx_v)
    pltpu.sync_copy(ctx_idx.at[pl.ds(wid * BPW, BPW)], ctx_idx_v)
    pltpu.sync_copy(neg_idx.at[pl.ds(wid * BPW * NNEG, BPW * NNEG)],
                    neg_raw_v)

    iota = lax.iota(jnp.int32, L)

    def chunk_body(c, carry):
        base = c * (CH * NNEG)
        for j in range(NNEG):
            for g in range(GPC):
                fidx = base + (iota + g * L) * NNEG + j
                col = plsc.load_gather(neg_raw_v, [fidx])
                neg_idx_t[j, pl.ds(g * L, L)] = col >> 1
                npar_t[j, pl.ds(g * L, L)] = (col & 1) * D

        for g in range(GPC):
            sl = pl.ds(c * CH + g * L, L)
            neg_idx_t[NNEG, pl.ds(g * L, L)] = in_idx_v[sl] >> 1
            neg_idx_t[NNEG + 1, pl.ds(g * L, L)] = ctx_idx_v[sl] >> 1

        for wave in range(NWAVE):
            j0 = wave * JH
            cps = []
            if wave == 0:
                cps.append(pltpu.async_copy(
                    in_table.at[neg_idx_t.at[NNEG]], in_rows, sem))
                cps.append(pltpu.async_copy(
                    out_table.at[neg_idx_t.at[NNEG + 1]], pos_rows, sem))
            for j in range(JH):
                cps.append(pltpu.async_copy(
                    out_table.at[neg_idx_t.at[j0 + j]], neg_rows.at[j], sem))
            for cp in cps:
                cp.wait()

            for g in range(GPC):
                rid = iota + (g * L)
                sl = pl.ds(c * CH + g * L, L)
                gsl = pl.ds(g * L, L)
                in_par = (in_idx_v[sl] & 1) * D
                ctx_par = (ctx_idx_v[sl] & 1) * D
                npars = [npar_t[j0 + j, gsl] for j in range(JH)]

                def d_pass(d, accs):
                    rot = (iota + d) & (D - 1)
                    inv = plsc.load_gather(in_rows, [rid, in_par + rot])
                    if wave == 0:
                        pv = plsc.load_gather(pos_rows,
                                              [rid, ctx_par + rot])
                        new = [accs[0] + inv * pv]
                    else:
                        new = []
                    for j in range(JH):
                        jvec = jnp.full((L,), j, jnp.int32)
                        nv = plsc.load_gather(neg_rows,
                                              [jvec, rid, npars[j] + rot])
                        new.append(accs[j + (1 if wave == 0 else 0)] +
                                   inv * nv)
                    return tuple(new)

                nacc = JH + 1 if wave == 0 else JH
                zeros = tuple(jnp.zeros((L,), jnp.float32)
                              for _ in range(nacc))
                accs = lax.fori_loop(0, D, d_pass, zeros)

                off = c * CH + g * L
                if wave == 0:
                    pos_v[pl.ds(off, L)] = accs[0]
                    accs = accs[1:]
                widx = (iota + off) * NNEG
                for j in range(JH):
                    plsc.store_scatter(neg_v, [widx + (j0 + j)], accs[j])
        return carry

    lax.fori_loop(0, NCHUNK, chunk_body, 0)

    pltpu.sync_copy(pos_v, pos_out.at[pl.ds(wid * BPW, BPW)])
    pltpu.sync_copy(neg_v, neg_out.at[pl.ds(wid * BPW * NNEG, BPW * NNEG)])


@jax.jit
def _skipgram(in_table, out_table, in_idx, ctx_idx, neg_idx):
    mesh = plsc.VectorSubcoreMesh(core_axis_name="c", subcore_axis_name="s")
    f = pl.kernel(
        _body,
        out_type=[
            jax.ShapeDtypeStruct((B,), jnp.float32),
            jax.ShapeDtypeStruct((B * NNEG,), jnp.float32),
        ],
        mesh=mesh,
        scratch_types=[
            pltpu.VMEM((BPW,), jnp.int32),
            pltpu.VMEM((BPW,), jnp.int32),
            pltpu.VMEM((BPW * NNEG,), jnp.int32),
            pltpu.VMEM((NNEG + 2, CH), jnp.int32),
            pltpu.VMEM((NNEG, CH), jnp.int32),
            pltpu.VMEM((CH, DP), jnp.float32),
            pltpu.VMEM((CH, DP), jnp.float32),
            pltpu.VMEM((JH, CH, DP), jnp.float32),
            pltpu.VMEM((BPW,), jnp.float32),
            pltpu.VMEM((BPW * NNEG,), jnp.float32),
            pltpu.SemaphoreType.DMA,
        ],
        compiler_params=pltpu.CompilerParams(needs_layout_passes=False),
    )
    return f(in_table, out_table, in_idx, ctx_idx, neg_idx)


BR = 10000


def _depad_body(x1_ref, x2_ref, o1_ref, o2_ref, xbuf, sem):
    i = pl.program_id(0)
    for k, (x_ref, o_ref) in enumerate(((x1_ref, o1_ref), (x2_ref, o2_ref))):
        pltpu.make_async_copy(x_ref.at[pl.ds(i * BR, BR), :], xbuf.at[k],
                              sem).start()
    for k, (x_ref, o_ref) in enumerate(((x1_ref, o1_ref), (x2_ref, o2_ref))):
        pltpu.make_async_copy(x_ref.at[pl.ds(i * BR, BR), :], xbuf.at[k],
                              sem).wait()
        x3 = xbuf[k].reshape(BR // 2, 2, D)
        o_ref[...] = jnp.concatenate([x3[:, 0, :], x3[:, 1, :]], axis=1)


@jax.jit
def _depad(t1, t2):
    v = t1.shape[0]
    grid = (v // BR,)
    spec_in = pl.BlockSpec(memory_space=pl.ANY)
    spec_out = pl.BlockSpec((BR // 2, DP), lambda i: (i, 0))
    return pl.pallas_call(
        _depad_body,
        grid=grid,
        in_specs=[spec_in, spec_in],
        out_specs=[spec_out, spec_out],
        out_shape=[
            jax.ShapeDtypeStruct((v // 2, DP), jnp.float32),
            jax.ShapeDtypeStruct((v // 2, DP), jnp.float32),
        ],
        scratch_shapes=[
            pltpu.VMEM((2, BR, D), jnp.float32),
            pltpu.SemaphoreType.DMA,
        ],
    )(t1, t2)


def kernel(in_table, out_table, inputs, contexts, negatives):
    in_t2, out_t2 = _depad(in_table, out_table)
    in_idx = inputs.reshape(B)
    ctx_idx = contexts.reshape(B)
    neg_idx = negatives.reshape(B * NNEG)
    pos, neg = _skipgram(in_t2, out_t2, in_idx, ctx_idx, neg_idx)
    return pos, neg.reshape(B, NNEG)


# speedup vs baseline: 1.5902x; 1.5902x over previous
import jax
import jax.numpy as jnp
from jax import lax
from jax.experimental import pallas as pl
from jax.experimental.pallas import tpu as pltpu, tpu_sc as plsc

B = 16384
D = 64
NNEG = 20
NC = 2
NS = 16
NW = NC * NS
BPW = B // NW
CH = 32
NCHUNK = BPW // CH
L = 16
GPC = CH // L


def _body(in_table, out_table, in_idx, ctx_idx, neg_idx, pos_out, neg_out,
          in_idx_v, ctx_idx_v, neg_raw_v, neg_idx_t, in_rows, pos_rows,
          neg_rows, pos_v, neg_v, sem):
    wid = lax.axis_index("s") * NC + lax.axis_index("c")

    pltpu.sync_copy(in_idx.at[wid], in_idx_v)
    pltpu.sync_copy(ctx_idx.at[wid], ctx_idx_v)
    pltpu.sync_copy(neg_idx.at[wid], neg_raw_v)

    iota = lax.iota(jnp.int32, L)

    def chunk_body(c, carry):
        cvec = jnp.zeros((L,), jnp.int32) + c
        for j in range(NNEG):
            jvec = jnp.full((L,), j, jnp.int32)
            for g in range(GPC):
                rid = iota + (g * L)
                col = plsc.load_gather(neg_raw_v, [cvec, rid, jvec])
                neg_idx_t[j, pl.ds(g * L, L)] = col

        cps = [
            pltpu.async_copy(in_table.at[in_idx_v.at[c]], in_rows, sem),
            pltpu.async_copy(out_table.at[ctx_idx_v.at[c]], pos_rows, sem),
        ]
        for j in range(NNEG):
            cps.append(pltpu.async_copy(out_table.at[neg_idx_t.at[j]],
                                        neg_rows.at[j], sem))
        for cp in cps:
            cp.wait()

        for g in range(GPC):
            rid = iota + (g * L)

            def d_body(d, accs):
                dvec = (iota + d) & (D - 1)
                inv = plsc.load_gather(in_rows, [rid, dvec])
                pv = plsc.load_gather(pos_rows, [rid, dvec])
                new = [accs[0] + inv * pv]
                for j in range(NNEG):
                    jvec = jnp.full((L,), j, jnp.int32)
                    nv = plsc.load_gather(neg_rows, [jvec, rid, dvec])
                    new.append(accs[j + 1] + inv * nv)
                return tuple(new)

            zeros = tuple(jnp.zeros((L,), jnp.float32)
                          for _ in range(NNEG + 1))
            accs = lax.fori_loop(0, D, d_body, zeros)

            off = c * CH + g * L
            pos_v[pl.ds(off, L)] = accs[0]
            rid_w = iota + off
            for j in range(NNEG):
                jvec = jnp.full((L,), j, jnp.int32)
                plsc.store_scatter(neg_v, [rid_w, jvec], accs[j + 1])
        return carry

    lax.fori_loop(0, NCHUNK, chunk_body, 0)

    pltpu.sync_copy(pos_v, pos_out.at[wid])
    pltpu.sync_copy(neg_v, neg_out.at[wid])


@jax.jit
def _skipgram(in_table, out_table, in_idx, ctx_idx, neg_idx):
    mesh = plsc.VectorSubcoreMesh(core_axis_name="c", subcore_axis_name="s")
    f = pl.kernel(
        _body,
        out_type=[
            jax.ShapeDtypeStruct((NW, BPW), jnp.float32),
            jax.ShapeDtypeStruct((NW, BPW, NNEG), jnp.float32),
        ],
        mesh=mesh,
        scratch_types=[
            pltpu.VMEM((NCHUNK, CH), jnp.int32),
            pltpu.VMEM((NCHUNK, CH), jnp.int32),
            pltpu.VMEM((NCHUNK, CH, NNEG), jnp.int32),
            pltpu.VMEM((NNEG, CH), jnp.int32),
            pltpu.VMEM((CH, D), jnp.float32),
            pltpu.VMEM((CH, D), jnp.float32),
            pltpu.VMEM((NNEG, CH, D), jnp.float32),
            pltpu.VMEM((BPW,), jnp.float32),
            pltpu.VMEM((BPW, NNEG), jnp.float32),
            pltpu.SemaphoreType.DMA,
        ],
        compiler_params=pltpu.CompilerParams(use_tc_tiling_on_sc=False,
                                             needs_layout_passes=False),
    )
    return f(in_table, out_table, in_idx, ctx_idx, neg_idx)


def kernel(in_table, out_table, inputs, contexts, negatives):
    in_idx = inputs.reshape(NW, NCHUNK, CH)
    ctx_idx = contexts.reshape(NW, NCHUNK, CH)
    neg_idx = negatives.reshape(NW, NCHUNK, CH, NNEG)
    pos, neg = _skipgram(in_table, out_table, in_idx, ctx_idx, neg_idx)
    return pos.reshape(B), neg.reshape(B, NNEG)


# speedup vs baseline: 1.6162x; 1.0163x over previous
import jax
import jax.numpy as jnp
from jax import lax
from jax.experimental import pallas as pl
from jax.experimental.pallas import tpu as pltpu, tpu_sc as plsc

B = 16384
D = 64
NNEG = 20
NC = 2
NS = 16
NW = NC * NS
BPW = B // NW
CH = 32
NCHUNK = BPW // CH
L = 16
GPC = CH // L


def _body(in_table, out_table, in_idx, ctx_idx, neg_idx, pos_out, neg_out,
          in_idx_v, ctx_idx_v, neg_raw_v, neg_idx_t, in_rows, pos_rows,
          neg_rows, pos_v, neg_v, sems):
    wid = lax.axis_index("s") * NC + lax.axis_index("c")

    pltpu.sync_copy(in_idx.at[wid], in_idx_v)
    pltpu.sync_copy(ctx_idx.at[wid], ctx_idx_v)
    pltpu.sync_copy(neg_idx.at[wid], neg_raw_v)

    iota = lax.iota(jnp.int32, L)

    def stage(c, p):
        cvec = jnp.zeros((L,), jnp.int32) + c
        for j in range(NNEG):
            jvec = jnp.full((L,), j, jnp.int32)
            for g in range(GPC):
                rid = iota + (g * L)
                col = plsc.load_gather(neg_raw_v, [cvec, rid, jvec])
                neg_idx_t[p, j, pl.ds(g * L, L)] = col
        cps = [
            pltpu.async_copy(in_table.at[in_idx_v.at[c]], in_rows.at[p],
                             sems.at[p]),
            pltpu.async_copy(out_table.at[ctx_idx_v.at[c]], pos_rows.at[p],
                             sems.at[p]),
        ]
        for j in range(NNEG):
            cps.append(pltpu.async_copy(out_table.at[neg_idx_t.at[p, j]],
                                        neg_rows.at[p, j], sems.at[p]))
        return cps

    def drain(p):
        pltpu.make_async_copy(in_table.at[in_idx_v.at[0]], in_rows.at[p],
                              sems.at[p]).wait()
        pltpu.make_async_copy(out_table.at[ctx_idx_v.at[0]], pos_rows.at[p],
                              sems.at[p]).wait()
        for j in range(NNEG):
            pltpu.make_async_copy(out_table.at[neg_idx_t.at[p, j]],
                                  neg_rows.at[p, j], sems.at[p]).wait()

    stage(0, 0)

    def chunk_body(c, carry):
        p = c & 1
        drain(p)

        @pl.when(c + 1 < NCHUNK)
        def _():
            stage(c + 1, 1 - p)

        pvec = jnp.zeros((L,), jnp.int32) + p

        for g in range(GPC):
            rid = iota + (g * L)

            def d_body(d, accs):
                dvec = (iota + d) & (D - 1)
                inv = plsc.load_gather(in_rows, [pvec, rid, dvec])
                pv = plsc.load_gather(pos_rows, [pvec, rid, dvec])
                new = [accs[0] + inv * pv]
                for j in range(NNEG):
                    jvec = jnp.full((L,), j, jnp.int32)
                    nv = plsc.load_gather(neg_rows, [pvec, jvec, rid, dvec])
                    new.append(accs[j + 1] + inv * nv)
                return tuple(new)

            zeros = tuple(jnp.zeros((L,), jnp.float32)
                          for _ in range(NNEG + 1))
            accs = lax.fori_loop(0, D, d_body, zeros)

            off = c * CH + g * L
            pos_v[pl.ds(off, L)] = accs[0]
            rid_w = iota + off
            for j in range(NNEG):
                jvec = jnp.full((L,), j, jnp.int32)
                plsc.store_scatter(neg_v, [rid_w, jvec], accs[j + 1])
        return carry

    lax.fori_loop(0, NCHUNK, chunk_body, 0)

    pltpu.sync_copy(pos_v, pos_out.at[wid])
    pltpu.sync_copy(neg_v, neg_out.at[wid])


@jax.jit
def _skipgram(in_table, out_table, in_idx, ctx_idx, neg_idx):
    mesh = plsc.VectorSubcoreMesh(core_axis_name="c", subcore_axis_name="s")
    f = pl.kernel(
        _body,
        out_type=[
            jax.ShapeDtypeStruct((NW, BPW), jnp.float32),
            jax.ShapeDtypeStruct((NW, BPW, NNEG), jnp.float32),
        ],
        mesh=mesh,
        scratch_types=[
            pltpu.VMEM((NCHUNK, CH), jnp.int32),
            pltpu.VMEM((NCHUNK, CH), jnp.int32),
            pltpu.VMEM((NCHUNK, CH, NNEG), jnp.int32),
            pltpu.VMEM((2, NNEG, CH), jnp.int32),
            pltpu.VMEM((2, CH, D), jnp.float32),
            pltpu.VMEM((2, CH, D), jnp.float32),
            pltpu.VMEM((2, NNEG, CH, D), jnp.float32),
            pltpu.VMEM((BPW,), jnp.float32),
            pltpu.VMEM((BPW, NNEG), jnp.float32),
            pltpu.SemaphoreType.DMA((2,)),
        ],
        compiler_params=pltpu.CompilerParams(use_tc_tiling_on_sc=False,
                                             needs_layout_passes=False),
    )
    return f(in_table, out_table, in_idx, ctx_idx, neg_idx)


def kernel(in_table, out_table, inputs, contexts, negatives):
    in_idx = inputs.reshape(NW, NCHUNK, CH)
    ctx_idx = contexts.reshape(NW, NCHUNK, CH)
    neg_idx = negatives.reshape(NW, NCHUNK, CH, NNEG)
    pos, neg = _skipgram(in_table, out_table, in_idx, ctx_idx, neg_idx)
    return pos.reshape(B), neg.reshape(B, NNEG)
